# baseline (speedup 1.0000x reference)
import functools

import jax
import jax.numpy as jnp
from jax import lax
from jax.experimental import pallas as pl
from jax.experimental.pallas import tpu as pltpu
from jax.experimental.pallas import tpu_sc as plsc

N = 10000
N_PAD = 10240
D_IN = 256
D_HID = 256
D_OUT = 64
E = 160000
EB = 128
NS = 16
NC = 2
NT = NC * NS
E_PAD = NT * 5120
DEAD = 10016
NH = N_PAD // 2
ACC_R = NH + 64
CROWS = 48
LW = CROWS * EB
BM = 2048

_EPT = E_PAD // NT

_SC_PARAMS = pltpu.CompilerParams(needs_layout_passes=False)


@functools.cache
def _get_prep_kernel():
    mesh = plsc.VectorSubcoreMesh(core_axis_name="c", subcore_axis_name="s",
                                  num_cores=NC, num_subcores=NS)
    return functools.partial(
        pl.kernel,
        out_type=(jax.ShapeDtypeStruct((NT, N_PAD), jnp.float32),
                  jax.ShapeDtypeStruct((NT * LW,), jnp.int32),
                  jax.ShapeDtypeStruct((NT * LW,), jnp.int32),
                  jax.ShapeDtypeStruct((NT * LW,), jnp.int32),
                  jax.ShapeDtypeStruct((NT * LW,), jnp.int32),
                  jax.ShapeDtypeStruct((NT, 16), jnp.int32)),
        mesh=mesh,
        scratch_types=[
            pltpu.VMEM((_EPT,), jnp.int32),
            pltpu.VMEM((_EPT,), jnp.int32),
            pltpu.VMEM((N_PAD,), jnp.float32),
            pltpu.VMEM((LW,), jnp.int32),
            pltpu.VMEM((LW,), jnp.int32),
            pltpu.VMEM((LW,), jnp.int32),
            pltpu.VMEM((LW,), jnp.int32),
            pltpu.VMEM((16,), jnp.int32),
        ],
        compiler_params=_SC_PARAMS,
    )(_prep_body)


def _prep_body(src_hbm, dst_hbm, deg_hbm, losrc_hbm, lodst_hbm, hisrc_hbm,
               hidst_hbm, cnt_hbm, srcv, dstv, degl, lsrc, ldst, hsrc, hdst,
               cntv):
    c = lax.axis_index("c")
    s = lax.axis_index("s")
    t = c * NS + s
    zero16f = jnp.zeros((16,), jnp.float32)
    zero16i = jnp.zeros((16,), jnp.int32)
    dead16 = jnp.full((16,), DEAD, jnp.int32)

    def zrow(i, _):
        degl[pl.ds(i * 16, 16)] = zero16f
        return 0

    lax.fori_loop(0, N_PAD // 16, zrow, 0)

    def fill(i, _):
        lsrc[pl.ds(i * 16, 16)] = zero16i
        ldst[pl.ds(i * 16, 16)] = dead16
        hsrc[pl.ds(i * 16, 16)] = zero16i
        hdst[pl.ds(i * 16, 16)] = dead16
        return 0

    lax.fori_loop(0, LW // 16, fill, 0)

    pltpu.sync_copy(src_hbm.at[pl.ds(t * _EPT, _EPT)], srcv)
    pltpu.sync_copy(dst_hbm.at[pl.ds(t * _EPT, _EPT)], dstv)

    ones = jnp.ones((16,), jnp.float32)

    def acc(i, carry):
        off_lo, off_hi = carry
        s16 = srcv[pl.ds(i * 16, 16)]
        d16 = dstv[pl.ds(i * 16, 16)]
        plsc.addupdate_scatter(degl, [d16], ones)
        mlo = d16 < NH
        mhi = jnp.logical_not(mlo)
        plsc.store_compressed(lsrc.at[pl.ds(off_lo, 16)], s16, mask=mlo)
        plsc.store_compressed(ldst.at[pl.ds(off_lo, 16)], d16, mask=mlo)
        plsc.store_compressed(hsrc.at[pl.ds(off_hi, 16)], s16, mask=mhi)
        plsc.store_compressed(hdst.at[pl.ds(off_hi, 16)], d16, mask=mhi)
        pc = plsc.all_reduce_population_count(mlo)[0]
        return (off_lo + pc, off_hi + (16 - pc))

    off_lo, off_hi = lax.fori_loop(0, _EPT // 16, acc, (0, 0))

    nb_lo = lax.shift_right_logical(off_lo + 127, 7)
    nb_hi = lax.shift_right_logical(off_hi + 127, 7)
    iota = lax.iota(jnp.int32, 16)
    cntv[...] = jnp.where(iota == 0, nb_lo, jnp.where(iota == 1, nb_hi, 0))

    pltpu.sync_copy(degl, deg_hbm.at[t])
    pltpu.sync_copy(lsrc, losrc_hbm.at[pl.ds(t * LW, LW)])
    pltpu.sync_copy(ldst, lodst_hbm.at[pl.ds(t * LW, LW)])
    pltpu.sync_copy(hsrc, hisrc_hbm.at[pl.ds(t * LW, LW)])
    pltpu.sync_copy(hdst, hidst_hbm.at[pl.ds(t * LW, LW)])
    pltpu.sync_copy(cntv, cnt_hbm.at[t])


@functools.cache
def _get_agg_kernel():
    mesh = plsc.VectorSubcoreMesh(core_axis_name="c", subcore_axis_name="s",
                                  num_cores=NC, num_subcores=NS)
    rpt = NH // NS

    @functools.partial(
        pl.kernel,
        out_type=(jax.ShapeDtypeStruct((N_PAD, 128), jnp.float32),
                  jax.ShapeDtypeStruct((N_PAD, 128), jnp.float32)),
        mesh=mesh,
        scratch_types=[
            pltpu.VMEM((16,), jnp.int32),
            pltpu.VMEM((NT, 16), jnp.int32),
            pltpu.VMEM((CROWS, EB), jnp.int32),
            pltpu.VMEM((CROWS, EB), jnp.int32),
            pltpu.VMEM((2, EB), jnp.int32),
            pltpu.VMEM((EB, 128), jnp.float32),
            pltpu.VMEM((EB, 128), jnp.float32),
            pltpu.VMEM((128, 128), jnp.float32),
            pltpu.VMEM_SHARED((ACC_R, 128), jnp.float32),
            pltpu.SemaphoreType.DMA,
            pltpu.SemaphoreType.DMA,
            pltpu.SemaphoreType.DMA,
            pltpu.SemaphoreType.DMA,
            pltpu.SemaphoreType.DMA,
        ],
        compiler_params=_SC_PARAMS,
    )
    def agg(h0_hbm, h1_hbm, losrc_hbm, lodst_hbm, hisrc_hbm, hidst_hbm,
            cnt_hbm, par_hbm, out0_hbm, out1_hbm,
            par_v, cnt_v, src_v, dst_v, idx_v, buf_a, buf_b, zbuf, acc,
            sem_a, sem_b, sem_z, sem_sa, sem_sb):
        c = lax.axis_index("c")
        s = lax.axis_index("s")
        zero16 = jnp.zeros((16,), jnp.float32)

        pltpu.sync_copy(par_hbm, par_v)
        pltpu.sync_copy(cnt_hbm, cnt_v)

        def zrow(i, _):
            for q in range(8):
                zbuf[i, pl.ds(q * 16, 16)] = zero16
            return 0

        lax.fori_loop(0, 128, zrow, 0)

        nchunk = par_v[...][0]
        ch_base = jnp.where(nchunk == 2, 2 * s, 16 * c + s)
        arpt = ACC_R // NS

        def remap(dref, g, base, row):
            for q in range(8):
                d = dref[g, pl.ds(q * 16, 16)]
                ok = jnp.logical_and(d >= base, d < base + NH)
                idx_v[row, pl.ds(q * 16, 16)] = jnp.where(ok, d - base, NH)

        def chunk(h_hbm, sl_hbm, dl_hbm, k, p):
            ch = ch_base + k
            rowb = pl.multiple_of(ch * CROWS, 8)
            pltpu.sync_copy(sl_hbm.at[pl.ds(rowb, CROWS)], src_v)
            pltpu.sync_copy(dl_hbm.at[pl.ds(rowb, CROWS)], dst_v)
            nblk = cnt_v[ch, pl.ds(0, 16)][p]
            base = p * NH

            @pl.when(nblk > 0)
            def _():
                pltpu.async_copy(h_hbm.at[src_v.at[0]], buf_a, sem_a)

            @pl.when(nblk > 1)
            def _():
                pltpu.async_copy(h_hbm.at[src_v.at[1]], buf_b, sem_b)

            def body(i, _):
                g0 = 2 * i
                g1 = 2 * i + 1
                pltpu.make_async_copy(h_hbm.at[src_v.at[g0]], buf_a,
                                      sem_a).wait()
                remap(dst_v, g0, base, 0)
                pltpu.async_copy(buf_a, acc.at[idx_v.at[0]], sem_sa, add=True)
                pltpu.make_async_copy(h_hbm.at[src_v.at[g1]], buf_b,
                                      sem_b).wait()
                remap(dst_v, g1, base, 1)
                pltpu.async_copy(buf_b, acc.at[idx_v.at[1]], sem_sb, add=True)
                pltpu.make_async_copy(buf_a, acc.at[idx_v.at[0]],
                                      sem_sa).wait()

                @pl.when(g0 + 2 < nblk)
                def _():
                    pltpu.async_copy(h_hbm.at[src_v.at[g0 + 2]], buf_a, sem_a)

                pltpu.make_async_copy(buf_b, acc.at[idx_v.at[1]],
                                      sem_sb).wait()

                @pl.when(g1 + 2 < nblk)
                def _():
                    pltpu.async_copy(h_hbm.at[src_v.at[g1 + 2]], buf_b, sem_b)

                return 0

            lax.fori_loop(0, lax.div(nblk, 2), body, 0)

            @pl.when(lax.rem(nblk, 2) == 1)
            def _():
                g = nblk - 1
                pltpu.make_async_copy(h_hbm.at[src_v.at[g]], buf_a,
                                      sem_a).wait()
                remap(dst_v, g, base, 0)
                pltpu.sync_copy(buf_a, acc.at[idx_v.at[0]], add=True)

        for p, (sl_hbm, dl_hbm) in enumerate(
                ((losrc_hbm, lodst_hbm), (hisrc_hbm, hidst_hbm))):
            for r, n in ((0, 128), (128, 128), (256, arpt - 256)):
                pltpu.async_copy(zbuf.at[pl.ds(0, n)],
                                 acc.at[pl.ds(s * arpt + r, n)], sem_z)
            for r, n in ((0, 128), (128, 128), (256, arpt - 256)):
                pltpu.make_async_copy(zbuf.at[pl.ds(0, n)],
                                      acc.at[pl.ds(s * arpt + r, n)],
                                      sem_z).wait()
            plsc.subcore_barrier()

            for k in range(2):
                @pl.when(jnp.logical_and(k < nchunk, c == 0))
                def _(k=k, p=p):
                    chunk(h0_hbm, sl_hbm, dl_hbm, k, p)

                @pl.when(jnp.logical_and(k < nchunk, c == 1))
                def _(k=k, p=p):
                    chunk(h1_hbm, sl_hbm, dl_hbm, k, p)

            plsc.subcore_barrier()

            @pl.when(c == 0)
            def _(p=p):
                pltpu.sync_copy(acc.at[pl.ds(s * rpt, rpt)],
                                out0_hbm.at[pl.ds(p * NH + s * rpt, rpt)])

            @pl.when(c == 1)
            def _(p=p):
                pltpu.sync_copy(acc.at[pl.ds(s * rpt, rpt)],
                                out1_hbm.at[pl.ds(p * NH + s * rpt, rpt)])

            plsc.subcore_barrier()

    return agg


def _mm1_body(x_ref, w_ref, dinv_ref, h0_ref, h1_ref):
    h = jnp.dot(x_ref[...], w_ref[...], preferred_element_type=jnp.float32)
    hh = h * dinv_ref[...]
    h0_ref[...] = hh[:, :128]
    h1_ref[...] = hh[:, 128:]


def _mm2_body(s0_ref, s1_ref, h0_ref, h1_ref, dinv_ref, b1_ref, w2_ref,
              o_ref):
    d = dinv_ref[...]
    z = jnp.concatenate(
        [d * (s0_ref[...] + h0_ref[...]), d * (s1_ref[...] + h1_ref[...])],
        axis=1) + b1_ref[...]
    z = jnp.maximum(z, 0.0)
    h2 = jnp.dot(z, w2_ref[...], preferred_element_type=jnp.float32)
    hh = h2 * d
    o_ref[...] = jnp.concatenate(
        [hh, jnp.zeros((BM, 128 - D_OUT), jnp.float32)], axis=1)


def _fin_body(p0_ref, p1_ref, h2_ref, dinv_ref, b2_ref, out_ref):
    d = dinv_ref[...]
    o = d * (p0_ref[...][:, :D_OUT] + p1_ref[...][:, :D_OUT]
             + h2_ref[...][:, :D_OUT]) + b2_ref[...]
    m = jnp.max(o, axis=1, keepdims=True)
    e = jnp.exp(o - m)
    lse = jnp.log(jnp.sum(e, axis=1, keepdims=True))
    out_ref[...] = o - m - lse


def _row_spec(dcols):
    return pl.BlockSpec((BM, dcols), lambda i: (i, 0))


def _full_spec(shape):
    return pl.BlockSpec(shape, lambda i: tuple(0 for _ in shape))


_GRID = (N_PAD // BM,)

_mm1 = pl.pallas_call(
    _mm1_body,
    grid=_GRID,
    in_specs=[_row_spec(D_IN), _full_spec((D_IN, D_HID)), _row_spec(1)],
    out_specs=[_row_spec(128)] * 2,
    out_shape=[jax.ShapeDtypeStruct((N_PAD, 128), jnp.float32)] * 2,
)

_mm2 = pl.pallas_call(
    _mm2_body,
    grid=_GRID,
    in_specs=[_row_spec(128)] * 4 +
             [_row_spec(1), _full_spec((1, D_HID)), _full_spec((D_HID, D_OUT))],
    out_specs=_row_spec(128),
    out_shape=jax.ShapeDtypeStruct((N_PAD, 128), jnp.float32),
)

_fin = pl.pallas_call(
    _fin_body,
    grid=_GRID,
    in_specs=[_row_spec(128)] * 3 + [_row_spec(1), _full_spec((1, D_OUT))],
    out_specs=_row_spec(D_OUT),
    out_shape=jax.ShapeDtypeStruct((N_PAD, D_OUT), jnp.float32),
)


def kernel(x, edge_index, W1, b1, W2, b2):
    src = edge_index[0].astype(jnp.int32)
    dst = edge_index[1].astype(jnp.int32)
    pad = E_PAD - E
    src_p = jnp.concatenate([src, jnp.zeros((pad,), jnp.int32)])
    dst_p = jnp.concatenate([dst, jnp.full((pad,), DEAD, jnp.int32)])
    x_pad = jnp.pad(x, ((0, N_PAD - N), (0, 0)))

    deg_p, lsrc, ldst, hsrc, hdst, cnts = _get_prep_kernel()(src_p, dst_p)
    deg = deg_p.sum(axis=0) + 1.0
    dinv = lax.rsqrt(deg)[:, None]
    lsrc2d = lsrc.reshape(NT * CROWS, EB)
    ldst2d = ldst.reshape(NT * CROWS, EB)
    hsrc2d = hsrc.reshape(NT * CROWS, EB)
    hdst2d = hdst.reshape(NT * CROWS, EB)

    agg = _get_agg_kernel()

    def params(nchunk):
        return jnp.array([nchunk] + [0] * 15, jnp.int32)

    h1_0, h1_1 = _mm1(x_pad, W1, dinv)
    s1_0, s1_1 = agg(h1_0, h1_1, lsrc2d, ldst2d, hsrc2d, hdst2d,
                     cnts, params(2))
    h2 = _mm2(s1_0, s1_1, h1_0, h1_1, dinv, b1.reshape(1, D_HID), W2)
    p0, p1 = agg(h2, h2, lsrc2d, ldst2d, hsrc2d, hdst2d, cnts, params(1))
    out = _fin(p0, p1, h2, dinv, b2.reshape(1, D_OUT))
    return out[:N]


# speedup vs baseline: 1.0942x; 1.0942x over previous
import functools

import jax
import jax.numpy as jnp
from jax import lax
from jax.experimental import pallas as pl
from jax.experimental.pallas import tpu as pltpu
from jax.experimental.pallas import tpu_sc as plsc

N = 10000
N_PAD = 10240
D_IN = 256
D_HID = 256
D_OUT = 64
E = 160000
EB = 128
NS = 16
NC = 2
NT = NC * NS
NB = 32
BN = N_PAD // NB
FS = 384
LBW = 48 * EB
BM = 2048

_EPT = E // NT

_SC_PARAMS = pltpu.CompilerParams(needs_layout_passes=False)


def _bucket_of(d):
    return lax.shift_right_logical(
        lax.shift_right_logical(d, 6) * 205, 10)


@functools.cache
def _get_prep_kernel():
    mesh = plsc.VectorSubcoreMesh(core_axis_name="c", subcore_axis_name="s",
                                  num_cores=NC, num_subcores=NS)
    return functools.partial(
        pl.kernel,
        out_type=(jax.ShapeDtypeStruct((NT, N_PAD), jnp.float32),
                  jax.ShapeDtypeStruct((NT, NB * FS), jnp.int32),
                  jax.ShapeDtypeStruct((NT, NB * FS), jnp.int32),
                  jax.ShapeDtypeStruct((NT, NB), jnp.int32)),
        mesh=mesh,
        scratch_types=[
            pltpu.VMEM((_EPT + 16,), jnp.int32),
            pltpu.VMEM((_EPT + 16,), jnp.int32),
            pltpu.VMEM((N_PAD,), jnp.float32),
            pltpu.VMEM((NB * FS,), jnp.int32),
            pltpu.VMEM((NB * FS,), jnp.int32),
            pltpu.VMEM((NB,), jnp.int32),
        ],
        compiler_params=_SC_PARAMS,
    )(_prep_body)


def _prep_body(src_hbm, dst_hbm, deg_hbm, fsrc_hbm, fdst_hbm, cnt_hbm,
               srcv, dstv, degl, fsrc, fdst, cntv):
    c = lax.axis_index("c")
    s = lax.axis_index("s")
    t = c * NS + s
    zero16f = jnp.zeros((16,), jnp.float32)

    def zrow(i, _):
        degl[pl.ds(i * 16, 16)] = zero16f
        return 0

    lax.fori_loop(0, N_PAD // 16, zrow, 0)

    pltpu.sync_copy(src_hbm.at[pl.ds(t * _EPT, _EPT)],
                    srcv.at[pl.ds(0, _EPT)])
    pltpu.sync_copy(dst_hbm.at[pl.ds(t * _EPT, _EPT)],
                    dstv.at[pl.ds(0, _EPT)])

    ones = jnp.ones((16,), jnp.float32)
    iota16 = lax.iota(jnp.int32, 16)
    ntail = _EPT % 16

    def step(i, offs, tail_mask):
        s16 = srcv[pl.ds(i * 16, 16)]
        d16 = dstv[pl.ds(i * 16, 16)]
        plsc.addupdate_scatter(degl, [d16], ones, mask=tail_mask)
        bid = _bucket_of(d16)
        new = []
        for b in range(NB):
            m = bid == b
            if tail_mask is not None:
                m = jnp.logical_and(m, tail_mask)
            off = offs[b]
            plsc.store_compressed(fsrc.at[pl.ds(b * FS + off, 16)],
                                  s16, mask=m)
            plsc.store_compressed(fdst.at[pl.ds(b * FS + off, 16)],
                                  d16 - b * BN, mask=m)
            new.append(off + plsc.all_reduce_population_count(m)[0])
        return tuple(new)

    offs = lax.fori_loop(0, _EPT // 16,
                         lambda i, offs: step(i, offs, None),
                         tuple([0] * NB))
    offs = step(_EPT // 16, offs, iota16 < ntail)

    iota = lax.iota(jnp.int32, 16)
    for g in range(2):
        v = jnp.zeros((16,), jnp.int32)
        for j in range(16):
            v = jnp.where(iota == j, offs[g * 16 + j], v)
        cntv[pl.ds(g * 16, 16)] = v

    pltpu.sync_copy(degl, deg_hbm.at[t])
    pltpu.sync_copy(fsrc, fsrc_hbm.at[t])
    pltpu.sync_copy(fdst, fdst_hbm.at[t])
    pltpu.sync_copy(cntv, cnt_hbm.at[t])


@functools.cache
def _get_agg_kernel():
    mesh = plsc.VectorSubcoreMesh(core_axis_name="c", subcore_axis_name="s",
                                  num_cores=NC, num_subcores=NS)

    @functools.partial(
        pl.kernel,
        out_type=(jax.ShapeDtypeStruct((N_PAD, 128), jnp.float32),
                  jax.ShapeDtypeStruct((N_PAD, 128), jnp.float32)),
        mesh=mesh,
        scratch_types=[
            pltpu.VMEM((16,), jnp.int32),
            pltpu.VMEM((NT, NB), jnp.int32),
            pltpu.VMEM((1, NT), jnp.int32),
            pltpu.VMEM((NT, FS), jnp.int32),
            pltpu.VMEM((NT, FS), jnp.int32),
            pltpu.VMEM((LBW,), jnp.int32),
            pltpu.VMEM((LBW // EB, EB), jnp.int32),
            pltpu.VMEM((LBW,), jnp.int32),
            pltpu.VMEM((EB, 128), jnp.float32),
            pltpu.VMEM((EB, 128), jnp.float32),
            pltpu.VMEM((BN + 16, 128), jnp.float32),
            pltpu.SemaphoreType.DMA,
            pltpu.SemaphoreType.DMA,
            pltpu.SemaphoreType.DMA,
        ],
        compiler_params=_SC_PARAMS,
    )
    def agg(h0_hbm, h1_hbm, fsrc_hbm, fdst_hbm, cnt_hbm, par_hbm,
            out0_hbm, out1_hbm,
            par_v, cnt_v, fid_v, stg_s, stg_d, loc_s, loc_s2, loc_d,
            buf_a, buf_b, acc, sem_a, sem_b, sem_s):
        c = lax.axis_index("c")
        s = lax.axis_index("s")
        zero16 = jnp.zeros((16,), jnp.float32)
        iota = lax.iota(jnp.int32, 16)

        pltpu.sync_copy(par_hbm, par_v)
        pltpu.sync_copy(cnt_hbm, cnt_v)
        pv = par_v[...]
        nf = pv[0]
        f0 = jnp.where(c == 0, pv[1], pv[2])

        def addblock(buf, g):
            def col(j, _):
                idxv = loc_d[pl.ds(g * EB + j * 16, 16)]
                for lane in range(16):
                    r = idxv[lane]
                    e = j * 16 + lane
                    for q in range(8):
                        acc[r, pl.ds(q * 16, 16)] = (
                            acc[r, pl.ds(q * 16, 16)]
                            + buf[e, pl.ds(q * 16, 16)])
                return 0

            lax.fori_loop(0, 8, col, 0)

        def bucket(h_hbm, k):
            b = 2 * s + k

            def zrow(i, _):
                for q in range(8):
                    acc[i, pl.ds(q * 16, 16)] = zero16
                return 0

            lax.fori_loop(0, BN + 16, zrow, 0)

            fid_v[0, pl.ds(0, 16)] = (
                jnp.minimum(f0 + iota, NT - 1) * NB + b)
            fid_v[0, pl.ds(16, 16)] = (
                jnp.minimum(f0 + iota + 16, NT - 1) * NB + b)
            pltpu.async_copy(fsrc_hbm.at[fid_v.at[0]], stg_s, sem_s)
            pltpu.make_async_copy(fsrc_hbm.at[fid_v.at[0]], stg_s,
                                  sem_s).wait()
            pltpu.async_copy(fdst_hbm.at[fid_v.at[0]], stg_d, sem_s)
            pltpu.make_async_copy(fdst_hbm.at[fid_v.at[0]], stg_d,
                                  sem_s).wait()

            def frag(tt, off):
                crow = cnt_v[f0 + tt, pl.ds((b >> 4) * 16, 16)]
                ct = jnp.sum(jnp.where(iota == (b & 15), crow, 0))

                def vreg(j, o):
                    sv = stg_s[tt, pl.ds(j * 16, 16)]
                    dv = stg_d[tt, pl.ds(j * 16, 16)]
                    m = (j * 16 + iota) < ct
                    plsc.store_compressed(loc_s.at[pl.ds(o, 16)], sv, mask=m)
                    plsc.store_compressed(loc_d.at[pl.ds(o, 16)], dv, mask=m)
                    return o + plsc.all_reduce_population_count(m)[0]

                return lax.fori_loop(0, lax.shift_right_logical(ct + 15, 4),
                                     vreg, off)

            off = lax.fori_loop(0, nf, frag, 0)
            nblk = lax.shift_right_logical(off + 127, 7)

            def tail(j, _):
                pos = off + j * 16
                m = jnp.logical_and(pos + iota < nblk * EB,
                                    pos + iota >= off)
                plsc.store_compressed(loc_s.at[pl.ds(pos, 16)],
                                      jnp.zeros((16,), jnp.int32), mask=m)
                plsc.store_compressed(loc_d.at[pl.ds(pos, 16)],
                                      jnp.full((16,), BN, jnp.int32), mask=m)
                return 0

            lax.fori_loop(0, 8, tail, 0)

            def pack(g, _):
                for q in range(8):
                    loc_s2[g, pl.ds(q * 16, 16)] = (
                        loc_s[pl.ds(g * EB + q * 16, 16)])
                return 0

            lax.fori_loop(0, nblk, pack, 0)

            @pl.when(nblk > 0)
            def _():
                pltpu.async_copy(h_hbm.at[loc_s2.at[0]], buf_a, sem_a)

            @pl.when(nblk > 1)
            def _():
                pltpu.async_copy(h_hbm.at[loc_s2.at[1]], buf_b, sem_b)

            def body(i, _):
                g0 = 2 * i
                g1 = 2 * i + 1
                pltpu.make_async_copy(h_hbm.at[loc_s2.at[g0]], buf_a,
                                      sem_a).wait()
                addblock(buf_a, g0)

                @pl.when(g0 + 2 < nblk)
                def _():
                    pltpu.async_copy(h_hbm.at[loc_s2.at[g0 + 2]],
                                     buf_a, sem_a)

                pltpu.make_async_copy(h_hbm.at[loc_s2.at[g1]], buf_b,
                                      sem_b).wait()
                addblock(buf_b, g1)

                @pl.when(g1 + 2 < nblk)
                def _():
                    pltpu.async_copy(h_hbm.at[loc_s2.at[g1 + 2]],
                                     buf_b, sem_b)

                return 0

            lax.fori_loop(0, lax.div(nblk, 2), body, 0)

            @pl.when(lax.rem(nblk, 2) == 1)
            def _():
                g = nblk - 1
                pltpu.make_async_copy(h_hbm.at[loc_s2.at[g]], buf_a,
                                      sem_a).wait()
                addblock(buf_a, g)

            @pl.when(c == 0)
            def _():
                pltpu.sync_copy(acc.at[pl.ds(0, BN)],
                                out0_hbm.at[pl.ds(b * BN, BN)])

            @pl.when(c == 1)
            def _():
                pltpu.sync_copy(acc.at[pl.ds(0, BN)],
                                out1_hbm.at[pl.ds(b * BN, BN)])

        for k in range(2):
            @pl.when(c == 0)
            def _(k=k):
                bucket(h0_hbm, k)

            @pl.when(c == 1)
            def _(k=k):
                bucket(h1_hbm, k)

    return agg


def _mm1_body(x_ref, w_ref, dinv_ref, h0_ref, h1_ref):
    h = jnp.dot(x_ref[...], w_ref[...], preferred_element_type=jnp.float32)
    hh = h * dinv_ref[...]
    h0_ref[...] = hh[:, :128]
    h1_ref[...] = hh[:, 128:]


def _mm2_body(s0_ref, s1_ref, h0_ref, h1_ref, dinv_ref, b1_ref, w2_ref,
              o_ref):
    d = dinv_ref[...]
    z = jnp.concatenate(
        [d * (s0_ref[...] + h0_ref[...]), d * (s1_ref[...] + h1_ref[...])],
        axis=1) + b1_ref[...]
    z = jnp.maximum(z, 0.0)
    h2 = jnp.dot(z, w2_ref[...], preferred_element_type=jnp.float32)
    hh = h2 * d
    o_ref[...] = jnp.concatenate(
        [hh, jnp.zeros((BM, 128 - D_OUT), jnp.float32)], axis=1)


def _fin_body(p0_ref, p1_ref, h2_ref, dinv_ref, b2_ref, out_ref):
    d = dinv_ref[...]
    o = d * (p0_ref[...][:, :D_OUT] + p1_ref[...][:, :D_OUT]
             + h2_ref[...][:, :D_OUT]) + b2_ref[...]
    m = jnp.max(o, axis=1, keepdims=True)
    e = jnp.exp(o - m)
    lse = jnp.log(jnp.sum(e, axis=1, keepdims=True))
    out_ref[...] = o - m - lse


def _row_spec(dcols):
    return pl.BlockSpec((BM, dcols), lambda i: (i, 0))


def _full_spec(shape):
    return pl.BlockSpec(shape, lambda i: tuple(0 for _ in shape))


_GRID = (N_PAD // BM,)

_mm1 = pl.pallas_call(
    _mm1_body,
    grid=_GRID,
    in_specs=[_row_spec(D_IN), _full_spec((D_IN, D_HID)), _row_spec(1)],
    out_specs=[_row_spec(128)] * 2,
    out_shape=[jax.ShapeDtypeStruct((N_PAD, 128), jnp.float32)] * 2,
)

_mm2 = pl.pallas_call(
    _mm2_body,
    grid=_GRID,
    in_specs=[_row_spec(128)] * 4 +
             [_row_spec(1), _full_spec((1, D_HID)), _full_spec((D_HID, D_OUT))],
    out_specs=_row_spec(128),
    out_shape=jax.ShapeDtypeStruct((N_PAD, 128), jnp.float32),
)

_fin = pl.pallas_call(
    _fin_body,
    grid=_GRID,
    in_specs=[_row_spec(128)] * 3 + [_row_spec(1), _full_spec((1, D_OUT))],
    out_specs=_row_spec(D_OUT),
    out_shape=jax.ShapeDtypeStruct((N_PAD, D_OUT), jnp.float32),
)


def kernel(x, edge_index, W1, b1, W2, b2):
    src = edge_index[0].astype(jnp.int32)
    dst = edge_index[1].astype(jnp.int32)
    x_pad = jnp.pad(x, ((0, N_PAD - N), (0, 0)))

    deg_p, fsrc, fdst, cnts = _get_prep_kernel()(src, dst)
    fsrc = fsrc.reshape(NT * NB, FS)
    fdst = fdst.reshape(NT * NB, FS)
    deg = deg_p.sum(axis=0) + 1.0
    dinv = lax.rsqrt(deg)[:, None]

    agg = _get_agg_kernel()

    def params(nf, f0a, f0b):
        return jnp.array([nf, f0a, f0b] + [0] * 13, jnp.int32)

    h1_0, h1_1 = _mm1(x_pad, W1, dinv)
    s1_0, s1_1 = agg(h1_0, h1_1, fsrc, fdst, cnts, params(32, 0, 0))
    h2 = _mm2(s1_0, s1_1, h1_0, h1_1, dinv, b1.reshape(1, D_HID), W2)
    p0, p1 = agg(h2, h2, fsrc, fdst, cnts, params(16, 0, 16))
    out = _fin(p0, p1, h2, dinv, b2.reshape(1, D_OUT))
    return out[:N]


# speedup vs baseline: 1.2920x; 1.1808x over previous
import functools

import jax
import jax.numpy as jnp
from jax import lax
from jax.experimental import pallas as pl
from jax.experimental.pallas import tpu as pltpu
from jax.experimental.pallas import tpu_sc as plsc

N = 10000
N_PAD = 10240
D_IN = 256
D_HID = 256
D_OUT = 64
E = 160000
EB = 128
NS = 16
NC = 2
NT = NC * NS
NB = 32
BN = N_PAD // NB
FS = 384
LBW = 48 * EB
BM = 2048

_EPT = E // NT

_SC_PARAMS = pltpu.CompilerParams(needs_layout_passes=False)


def _bucket_of(d):
    return lax.shift_right_logical(
        lax.shift_right_logical(d, 6) * 205, 10)


@functools.cache
def _get_prep_kernel():
    mesh = plsc.VectorSubcoreMesh(core_axis_name="c", subcore_axis_name="s",
                                  num_cores=NC, num_subcores=NS)
    return functools.partial(
        pl.kernel,
        out_type=(jax.ShapeDtypeStruct((NT, N_PAD), jnp.float32),
                  jax.ShapeDtypeStruct((NT, NB * FS), jnp.int32),
                  jax.ShapeDtypeStruct((NT, NB * FS), jnp.int32),
                  jax.ShapeDtypeStruct((NT, NB), jnp.int32)),
        mesh=mesh,
        scratch_types=[
            pltpu.VMEM((_EPT + 16,), jnp.int32),
            pltpu.VMEM((_EPT + 16,), jnp.int32),
            pltpu.VMEM((N_PAD,), jnp.float32),
            pltpu.VMEM((NB * FS,), jnp.int32),
            pltpu.VMEM((NB * FS,), jnp.int32),
            pltpu.VMEM((NB,), jnp.int32),
        ],
        compiler_params=_SC_PARAMS,
    )(_prep_body)


def _prep_body(src_hbm, dst_hbm, deg_hbm, fsrc_hbm, fdst_hbm, cnt_hbm,
               srcv, dstv, degl, fsrc, fdst, cntv):
    c = lax.axis_index("c")
    s = lax.axis_index("s")
    t = c * NS + s
    zero16f = jnp.zeros((16,), jnp.float32)

    def zrow(i, _):
        degl[pl.ds(i * 16, 16)] = zero16f
        return 0

    lax.fori_loop(0, N_PAD // 16, zrow, 0)

    pltpu.sync_copy(src_hbm.at[pl.ds(t * _EPT, _EPT)],
                    srcv.at[pl.ds(0, _EPT)])
    pltpu.sync_copy(dst_hbm.at[pl.ds(t * _EPT, _EPT)],
                    dstv.at[pl.ds(0, _EPT)])

    ones = jnp.ones((16,), jnp.float32)
    iota16 = lax.iota(jnp.int32, 16)
    ntail = _EPT % 16

    def step(i, offs, tail_mask):
        s16 = srcv[pl.ds(i * 16, 16)]
        d16 = dstv[pl.ds(i * 16, 16)]
        plsc.addupdate_scatter(degl, [d16], ones, mask=tail_mask)
        bid = _bucket_of(d16)
        new = []
        for b in range(NB):
            m = bid == b
            if tail_mask is not None:
                m = jnp.logical_and(m, tail_mask)
            off = offs[b]
            plsc.store_compressed(fsrc.at[pl.ds(b * FS + off, 16)],
                                  s16, mask=m)
            plsc.store_compressed(fdst.at[pl.ds(b * FS + off, 16)],
                                  d16 - b * BN, mask=m)
            new.append(off + plsc.all_reduce_population_count(m)[0])
        return tuple(new)

    offs = lax.fori_loop(0, _EPT // 16,
                         lambda i, offs: step(i, offs, None),
                         tuple([0] * NB))
    offs = step(_EPT // 16, offs, iota16 < ntail)

    iota = lax.iota(jnp.int32, 16)
    for g in range(2):
        v = jnp.zeros((16,), jnp.int32)
        for j in range(16):
            v = jnp.where(iota == j, offs[g * 16 + j], v)
        cntv[pl.ds(g * 16, 16)] = v

    pltpu.sync_copy(degl, deg_hbm.at[t])
    pltpu.sync_copy(fsrc, fsrc_hbm.at[t])
    pltpu.sync_copy(fdst, fdst_hbm.at[t])
    pltpu.sync_copy(cntv, cnt_hbm.at[t])


@functools.cache
def _get_agg_kernel():
    mesh = plsc.VectorSubcoreMesh(core_axis_name="c", subcore_axis_name="s",
                                  num_cores=NC, num_subcores=NS)

    @functools.partial(
        pl.kernel,
        out_type=(jax.ShapeDtypeStruct((N_PAD, 128), jnp.float32),
                  jax.ShapeDtypeStruct((N_PAD, 128), jnp.float32)),
        mesh=mesh,
        scratch_types=[
            pltpu.VMEM((16,), jnp.int32),
            pltpu.VMEM((NT, NB), jnp.int32),
            pltpu.VMEM((1, NT), jnp.int32),
            pltpu.VMEM((NT, FS), jnp.int32),
            pltpu.VMEM((NT, FS), jnp.int32),
            pltpu.VMEM((LBW,), jnp.int32),
            pltpu.VMEM((LBW // EB, EB), jnp.int32),
            pltpu.VMEM((LBW,), jnp.int32),
            pltpu.VMEM((EB, 128), jnp.float32),
            pltpu.VMEM((EB, 128), jnp.float32),
            pltpu.VMEM((BN + 16, 128), jnp.float32),
            pltpu.SemaphoreType.DMA,
            pltpu.SemaphoreType.DMA,
            pltpu.SemaphoreType.DMA,
        ],
        compiler_params=_SC_PARAMS,
    )
    def agg(h0_hbm, h1_hbm, fsrc_hbm, fdst_hbm, cnt_hbm, par_hbm,
            out0_hbm, out1_hbm,
            par_v, cnt_v, fid_v, stg_s, stg_d, loc_s, loc_s2, loc_d,
            buf_a, buf_b, acc, sem_a, sem_b, sem_s):
        c = lax.axis_index("c")
        s = lax.axis_index("s")
        zero16 = jnp.zeros((16,), jnp.float32)
        iota = lax.iota(jnp.int32, 16)

        pltpu.sync_copy(par_hbm, par_v)
        pltpu.sync_copy(cnt_hbm, cnt_v)
        pv = par_v[...]
        nf = pv[0]
        f0 = jnp.where(c == 0, pv[1], pv[2])

        def addblock(buf, g):
            def col(j, _):
                idxv = loc_d[pl.ds(g * EB + j * 16, 16)]
                for lane in range(16):
                    r = idxv[lane]
                    e = j * 16 + lane
                    for q in range(8):
                        plsc.addupdate(acc.at[r, pl.ds(q * 16, 16)],
                                       buf[e, pl.ds(q * 16, 16)])
                return 0

            lax.fori_loop(0, 8, col, 0)

        def bucket(h_hbm, k):
            b = 2 * s + k

            def zrow(i, _):
                for q in range(8):
                    acc[i, pl.ds(q * 16, 16)] = zero16
                return 0

            lax.fori_loop(0, BN + 16, zrow, 0)

            fid_v[0, pl.ds(0, 16)] = (
                jnp.minimum(f0 + iota, NT - 1) * NB + b)
            fid_v[0, pl.ds(16, 16)] = (
                jnp.minimum(f0 + iota + 16, NT - 1) * NB + b)
            pltpu.async_copy(fsrc_hbm.at[fid_v.at[0]], stg_s, sem_s)
            pltpu.make_async_copy(fsrc_hbm.at[fid_v.at[0]], stg_s,
                                  sem_s).wait()
            pltpu.async_copy(fdst_hbm.at[fid_v.at[0]], stg_d, sem_s)
            pltpu.make_async_copy(fdst_hbm.at[fid_v.at[0]], stg_d,
                                  sem_s).wait()

            def frag(tt, off):
                crow = cnt_v[f0 + tt, pl.ds((b >> 4) * 16, 16)]
                ct = jnp.sum(jnp.where(iota == (b & 15), crow, 0))

                def vreg(j, o):
                    sv = stg_s[tt, pl.ds(j * 16, 16)]
                    dv = stg_d[tt, pl.ds(j * 16, 16)]
                    m = (j * 16 + iota) < ct
                    plsc.store_compressed(loc_s.at[pl.ds(o, 16)], sv, mask=m)
                    plsc.store_compressed(loc_d.at[pl.ds(o, 16)], dv, mask=m)
                    return o + plsc.all_reduce_population_count(m)[0]

                return lax.fori_loop(0, lax.shift_right_logical(ct + 15, 4),
                                     vreg, off)

            off = lax.fori_loop(0, nf, frag, 0)
            nblk = lax.shift_right_logical(off + 127, 7)

            def tail(j, _):
                pos = off + j * 16
                m = jnp.logical_and(pos + iota < nblk * EB,
                                    pos + iota >= off)
                plsc.store_compressed(loc_s.at[pl.ds(pos, 16)],
                                      jnp.zeros((16,), jnp.int32), mask=m)
                plsc.store_compressed(loc_d.at[pl.ds(pos, 16)],
                                      jnp.full((16,), BN, jnp.int32), mask=m)
                return 0

            lax.fori_loop(0, 8, tail, 0)

            def pack(g, _):
                for q in range(8):
                    loc_s2[g, pl.ds(q * 16, 16)] = (
                        loc_s[pl.ds(g * EB + q * 16, 16)])
                return 0

            lax.fori_loop(0, nblk, pack, 0)

            @pl.when(nblk > 0)
            def _():
                pltpu.async_copy(h_hbm.at[loc_s2.at[0]], buf_a, sem_a)

            @pl.when(nblk > 1)
            def _():
                pltpu.async_copy(h_hbm.at[loc_s2.at[1]], buf_b, sem_b)

            def body(i, _):
                g0 = 2 * i
                g1 = 2 * i + 1
                pltpu.make_async_copy(h_hbm.at[loc_s2.at[g0]], buf_a,
                                      sem_a).wait()
                addblock(buf_a, g0)

                @pl.when(g0 + 2 < nblk)
                def _():
                    pltpu.async_copy(h_hbm.at[loc_s2.at[g0 + 2]],
                                     buf_a, sem_a)

                pltpu.make_async_copy(h_hbm.at[loc_s2.at[g1]], buf_b,
                                      sem_b).wait()
                addblock(buf_b, g1)

                @pl.when(g1 + 2 < nblk)
                def _():
                    pltpu.async_copy(h_hbm.at[loc_s2.at[g1 + 2]],
                                     buf_b, sem_b)

                return 0

            lax.fori_loop(0, lax.div(nblk, 2), body, 0)

            @pl.when(lax.rem(nblk, 2) == 1)
            def _():
                g = nblk - 1
                pltpu.make_async_copy(h_hbm.at[loc_s2.at[g]], buf_a,
                                      sem_a).wait()
                addblock(buf_a, g)

            @pl.when(c == 0)
            def _():
                pltpu.sync_copy(acc.at[pl.ds(0, BN)],
                                out0_hbm.at[pl.ds(b * BN, BN)])

            @pl.when(c == 1)
            def _():
                pltpu.sync_copy(acc.at[pl.ds(0, BN)],
                                out1_hbm.at[pl.ds(b * BN, BN)])

        for k in range(2):
            @pl.when(c == 0)
            def _(k=k):
                bucket(h0_hbm, k)

            @pl.when(c == 1)
            def _(k=k):
                bucket(h1_hbm, k)

    return agg


def _mm1_body(x_ref, w_ref, dinv_ref, h0_ref, h1_ref):
    h = jnp.dot(x_ref[...], w_ref[...], preferred_element_type=jnp.float32)
    hh = h * dinv_ref[...]
    h0_ref[...] = hh[:, :128]
    h1_ref[...] = hh[:, 128:]


def _mm2_body(s0_ref, s1_ref, h0_ref, h1_ref, dinv_ref, b1_ref, w2_ref,
              o_ref):
    d = dinv_ref[...]
    z = jnp.concatenate(
        [d * (s0_ref[...] + h0_ref[...]), d * (s1_ref[...] + h1_ref[...])],
        axis=1) + b1_ref[...]
    z = jnp.maximum(z, 0.0)
    h2 = jnp.dot(z, w2_ref[...], preferred_element_type=jnp.float32)
    hh = h2 * d
    o_ref[...] = jnp.concatenate(
        [hh, jnp.zeros((BM, 128 - D_OUT), jnp.float32)], axis=1)


def _fin_body(p0_ref, p1_ref, h2_ref, dinv_ref, b2_ref, out_ref):
    d = dinv_ref[...]
    o = d * (p0_ref[...][:, :D_OUT] + p1_ref[...][:, :D_OUT]
             + h2_ref[...][:, :D_OUT]) + b2_ref[...]
    m = jnp.max(o, axis=1, keepdims=True)
    e = jnp.exp(o - m)
    lse = jnp.log(jnp.sum(e, axis=1, keepdims=True))
    out_ref[...] = o - m - lse


def _row_spec(dcols):
    return pl.BlockSpec((BM, dcols), lambda i: (i, 0))


def _full_spec(shape):
    return pl.BlockSpec(shape, lambda i: tuple(0 for _ in shape))


_GRID = (N_PAD // BM,)

_mm1 = pl.pallas_call(
    _mm1_body,
    grid=_GRID,
    in_specs=[_row_spec(D_IN), _full_spec((D_IN, D_HID)), _row_spec(1)],
    out_specs=[_row_spec(128)] * 2,
    out_shape=[jax.ShapeDtypeStruct((N_PAD, 128), jnp.float32)] * 2,
)

_mm2 = pl.pallas_call(
    _mm2_body,
    grid=_GRID,
    in_specs=[_row_spec(128)] * 4 +
             [_row_spec(1), _full_spec((1, D_HID)), _full_spec((D_HID, D_OUT))],
    out_specs=_row_spec(128),
    out_shape=jax.ShapeDtypeStruct((N_PAD, 128), jnp.float32),
)

_fin = pl.pallas_call(
    _fin_body,
    grid=_GRID,
    in_specs=[_row_spec(128)] * 3 + [_row_spec(1), _full_spec((1, D_OUT))],
    out_specs=_row_spec(D_OUT),
    out_shape=jax.ShapeDtypeStruct((N_PAD, D_OUT), jnp.float32),
)


def kernel(x, edge_index, W1, b1, W2, b2):
    src = edge_index[0].astype(jnp.int32)
    dst = edge_index[1].astype(jnp.int32)
    x_pad = jnp.pad(x, ((0, N_PAD - N), (0, 0)))

    deg_p, fsrc, fdst, cnts = _get_prep_kernel()(src, dst)
    fsrc = fsrc.reshape(NT * NB, FS)
    fdst = fdst.reshape(NT * NB, FS)
    deg = deg_p.sum(axis=0) + 1.0
    dinv = lax.rsqrt(deg)[:, None]

    agg = _get_agg_kernel()

    def params(nf, f0a, f0b):
        return jnp.array([nf, f0a, f0b] + [0] * 13, jnp.int32)

    h1_0, h1_1 = _mm1(x_pad, W1, dinv)
    s1_0, s1_1 = agg(h1_0, h1_1, fsrc, fdst, cnts, params(32, 0, 0))
    h2 = _mm2(s1_0, s1_1, h1_0, h1_1, dinv, b1.reshape(1, D_HID), W2)
    p0, p1 = agg(h2, h2, fsrc, fdst, cnts, params(16, 0, 16))
    out = _fin(p0, p1, h2, dinv, b2.reshape(1, D_OUT))
    return out[:N]


# speedup vs baseline: 1.3866x; 1.0732x over previous
import functools

import jax
import jax.numpy as jnp
from jax import lax
from jax.experimental import pallas as pl
from jax.experimental.pallas import tpu as pltpu
from jax.experimental.pallas import tpu_sc as plsc

N = 10000
N_PAD = 10240
D_IN = 256
D_HID = 256
D_OUT = 64
E = 160000
EB = 128
NS = 16
NC = 2
NT = NC * NS
NB = 32
BN = N_PAD // NB
FS = 384
LBW = 48 * EB
BM = 2048

_EPT = E // NT

_SC_PARAMS = pltpu.CompilerParams(needs_layout_passes=False)


def _bucket_of(d):
    return lax.shift_right_logical(
        lax.shift_right_logical(d, 6) * 205, 10)


@functools.cache
def _get_prep_kernel():
    mesh = plsc.VectorSubcoreMesh(core_axis_name="c", subcore_axis_name="s",
                                  num_cores=NC, num_subcores=NS)
    return functools.partial(
        pl.kernel,
        out_type=(jax.ShapeDtypeStruct((NT, N_PAD), jnp.float32),
                  jax.ShapeDtypeStruct((NT, NB * FS), jnp.int32),
                  jax.ShapeDtypeStruct((NT, NB * FS), jnp.int32),
                  jax.ShapeDtypeStruct((NT, NB), jnp.int32)),
        mesh=mesh,
        scratch_types=[
            pltpu.VMEM((_EPT + 16,), jnp.int32),
            pltpu.VMEM((_EPT + 16,), jnp.int32),
            pltpu.VMEM((N_PAD,), jnp.float32),
            pltpu.VMEM((NB * FS,), jnp.int32),
            pltpu.VMEM((NB * FS,), jnp.int32),
            pltpu.VMEM((NB,), jnp.int32),
        ],
        compiler_params=_SC_PARAMS,
    )(_prep_body)


def _prep_body(src_hbm, dst_hbm, deg_hbm, fsrc_hbm, fdst_hbm, cnt_hbm,
               srcv, dstv, degl, fsrc, fdst, cntv):
    c = lax.axis_index("c")
    s = lax.axis_index("s")
    t = c * NS + s
    zero16f = jnp.zeros((16,), jnp.float32)

    def zrow(i, _):
        degl[pl.ds(i * 16, 16)] = zero16f
        return 0

    lax.fori_loop(0, N_PAD // 16, zrow, 0)

    pltpu.sync_copy(src_hbm.at[pl.ds(t * _EPT, _EPT)],
                    srcv.at[pl.ds(0, _EPT)])
    pltpu.sync_copy(dst_hbm.at[pl.ds(t * _EPT, _EPT)],
                    dstv.at[pl.ds(0, _EPT)])

    ones = jnp.ones((16,), jnp.float32)
    iota16 = lax.iota(jnp.int32, 16)
    ntail = _EPT % 16

    def step(i, offs, tail_mask):
        s16 = srcv[pl.ds(i * 16, 16)]
        d16 = dstv[pl.ds(i * 16, 16)]
        plsc.addupdate_scatter(degl, [d16], ones, mask=tail_mask)
        bid = _bucket_of(d16)
        new = []
        for b in range(NB):
            m = bid == b
            if tail_mask is not None:
                m = jnp.logical_and(m, tail_mask)
            off = offs[b]
            plsc.store_compressed(fsrc.at[pl.ds(b * FS + off, 16)],
                                  s16, mask=m)
            plsc.store_compressed(fdst.at[pl.ds(b * FS + off, 16)],
                                  d16 - b * BN, mask=m)
            new.append(off + plsc.all_reduce_population_count(m)[0])
        return tuple(new)

    offs = lax.fori_loop(0, _EPT // 16,
                         lambda i, offs: step(i, offs, None),
                         tuple([0] * NB))
    offs = step(_EPT // 16, offs, iota16 < ntail)

    iota = lax.iota(jnp.int32, 16)
    for g in range(2):
        v = jnp.zeros((16,), jnp.int32)
        for j in range(16):
            v = jnp.where(iota == j, offs[g * 16 + j], v)
        cntv[pl.ds(g * 16, 16)] = v

    pltpu.sync_copy(degl, deg_hbm.at[t])
    pltpu.sync_copy(fsrc, fsrc_hbm.at[t])
    pltpu.sync_copy(fdst, fdst_hbm.at[t])
    pltpu.sync_copy(cntv, cnt_hbm.at[t])


@functools.cache
def _get_agg_kernel(nq=8):
    mesh = plsc.VectorSubcoreMesh(core_axis_name="c", subcore_axis_name="s",
                                  num_cores=NC, num_subcores=NS)

    @functools.partial(
        pl.kernel,
        out_type=(jax.ShapeDtypeStruct((N_PAD, 128), jnp.float32),
                  jax.ShapeDtypeStruct((N_PAD, 128), jnp.float32)),
        mesh=mesh,
        scratch_types=[
            pltpu.VMEM((16,), jnp.int32),
            pltpu.VMEM((NT, NB), jnp.int32),
            pltpu.VMEM((1, NT), jnp.int32),
            pltpu.VMEM((NT, FS), jnp.int32),
            pltpu.VMEM((NT, FS), jnp.int32),
            pltpu.VMEM((LBW,), jnp.int32),
            pltpu.VMEM((LBW // EB, EB), jnp.int32),
            pltpu.VMEM((LBW,), jnp.int32),
            pltpu.VMEM((EB, 128), jnp.float32),
            pltpu.VMEM((EB, 128), jnp.float32),
            pltpu.VMEM((BN + 16, 128), jnp.float32),
            pltpu.SemaphoreType.DMA,
            pltpu.SemaphoreType.DMA,
            pltpu.SemaphoreType.DMA,
        ],
        compiler_params=_SC_PARAMS,
    )
    def agg(h0_hbm, h1_hbm, fsrc_hbm, fdst_hbm, cnt_hbm, par_hbm,
            out0_hbm, out1_hbm,
            par_v, cnt_v, fid_v, stg_s, stg_d, loc_s, loc_s2, loc_d,
            buf_a, buf_b, acc, sem_a, sem_b, sem_s):
        c = lax.axis_index("c")
        s = lax.axis_index("s")
        zero16 = jnp.zeros((16,), jnp.float32)
        iota = lax.iota(jnp.int32, 16)

        pltpu.sync_copy(par_hbm, par_v)
        pltpu.sync_copy(cnt_hbm, cnt_v)
        pv = par_v[...]
        nf = pv[0]
        f0 = jnp.where(c == 0, pv[1], pv[2])

        def addblock(buf, g):
            def col(j, _):
                idxv = loc_d[pl.ds(g * EB + j * 16, 16)]
                for lane in range(16):
                    r = idxv[lane]
                    e = j * 16 + lane
                    for q in range(nq):
                        plsc.addupdate(acc.at[r, pl.ds(q * 16, 16)],
                                       buf[e, pl.ds(q * 16, 16)])
                return 0

            lax.fori_loop(0, 8, col, 0)

        def bucket(h_hbm, k):
            b = 2 * s + k

            def zrow(i, _):
                for q in range(8):
                    acc[i, pl.ds(q * 16, 16)] = zero16
                return 0

            lax.fori_loop(0, BN + 16, zrow, 0)

            fid_v[0, pl.ds(0, 16)] = (
                jnp.minimum(f0 + iota, NT - 1) * NB + b)
            fid_v[0, pl.ds(16, 16)] = (
                jnp.minimum(f0 + iota + 16, NT - 1) * NB + b)
            pltpu.async_copy(fsrc_hbm.at[fid_v.at[0]], stg_s, sem_s)
            pltpu.make_async_copy(fsrc_hbm.at[fid_v.at[0]], stg_s,
                                  sem_s).wait()
            pltpu.async_copy(fdst_hbm.at[fid_v.at[0]], stg_d, sem_s)
            pltpu.make_async_copy(fdst_hbm.at[fid_v.at[0]], stg_d,
                                  sem_s).wait()

            def frag(tt, off):
                crow = cnt_v[f0 + tt, pl.ds((b >> 4) * 16, 16)]
                ct = jnp.sum(jnp.where(iota == (b & 15), crow, 0))

                def vreg(j, o):
                    sv = stg_s[tt, pl.ds(j * 16, 16)]
                    dv = stg_d[tt, pl.ds(j * 16, 16)]
                    m = (j * 16 + iota) < ct
                    plsc.store_compressed(loc_s.at[pl.ds(o, 16)], sv, mask=m)
                    plsc.store_compressed(loc_d.at[pl.ds(o, 16)], dv, mask=m)
                    return o + plsc.all_reduce_population_count(m)[0]

                return lax.fori_loop(0, lax.shift_right_logical(ct + 15, 4),
                                     vreg, off)

            off = lax.fori_loop(0, nf, frag, 0)
            nblk = lax.shift_right_logical(off + 127, 7)

            def tail(j, _):
                pos = off + j * 16
                m = jnp.logical_and(pos + iota < nblk * EB,
                                    pos + iota >= off)
                plsc.store_compressed(loc_s.at[pl.ds(pos, 16)],
                                      jnp.zeros((16,), jnp.int32), mask=m)
                plsc.store_compressed(loc_d.at[pl.ds(pos, 16)],
                                      jnp.full((16,), BN, jnp.int32), mask=m)
                return 0

            lax.fori_loop(0, 8, tail, 0)

            def pack(g, _):
                for q in range(8):
                    loc_s2[g, pl.ds(q * 16, 16)] = (
                        loc_s[pl.ds(g * EB + q * 16, 16)])
                return 0

            lax.fori_loop(0, nblk, pack, 0)

            @pl.when(nblk > 0)
            def _():
                pltpu.async_copy(h_hbm.at[loc_s2.at[0]], buf_a, sem_a)

            @pl.when(nblk > 1)
            def _():
                pltpu.async_copy(h_hbm.at[loc_s2.at[1]], buf_b, sem_b)

            def body(i, _):
                g0 = 2 * i
                g1 = 2 * i + 1
                pltpu.make_async_copy(h_hbm.at[loc_s2.at[g0]], buf_a,
                                      sem_a).wait()
                addblock(buf_a, g0)

                @pl.when(g0 + 2 < nblk)
                def _():
                    pltpu.async_copy(h_hbm.at[loc_s2.at[g0 + 2]],
                                     buf_a, sem_a)

                pltpu.make_async_copy(h_hbm.at[loc_s2.at[g1]], buf_b,
                                      sem_b).wait()
                addblock(buf_b, g1)

                @pl.when(g1 + 2 < nblk)
                def _():
                    pltpu.async_copy(h_hbm.at[loc_s2.at[g1 + 2]],
                                     buf_b, sem_b)

                return 0

            lax.fori_loop(0, lax.div(nblk, 2), body, 0)

            @pl.when(lax.rem(nblk, 2) == 1)
            def _():
                g = nblk - 1
                pltpu.make_async_copy(h_hbm.at[loc_s2.at[g]], buf_a,
                                      sem_a).wait()
                addblock(buf_a, g)

            @pl.when(c == 0)
            def _():
                pltpu.sync_copy(acc.at[pl.ds(0, BN)],
                                out0_hbm.at[pl.ds(b * BN, BN)])

            @pl.when(c == 1)
            def _():
                pltpu.sync_copy(acc.at[pl.ds(0, BN)],
                                out1_hbm.at[pl.ds(b * BN, BN)])

        for k in range(2):
            @pl.when(c == 0)
            def _(k=k):
                bucket(h0_hbm, k)

            @pl.when(c == 1)
            def _(k=k):
                bucket(h1_hbm, k)

    return agg


def _mm1_body(x_ref, w_ref, dinv_ref, h0_ref, h1_ref):
    h = jnp.dot(x_ref[...], w_ref[...], preferred_element_type=jnp.float32)
    hh = h * dinv_ref[...]
    h0_ref[...] = hh[:, :128]
    h1_ref[...] = hh[:, 128:]


def _mm2_body(s0_ref, s1_ref, h0_ref, h1_ref, dinv_ref, b1_ref, w2_ref,
              o_ref):
    d = dinv_ref[...]
    z = jnp.concatenate(
        [d * (s0_ref[...] + h0_ref[...]), d * (s1_ref[...] + h1_ref[...])],
        axis=1) + b1_ref[...]
    z = jnp.maximum(z, 0.0)
    h2 = jnp.dot(z, w2_ref[...], preferred_element_type=jnp.float32)
    hh = h2 * d
    o_ref[...] = jnp.concatenate(
        [hh, jnp.zeros((BM, 128 - D_OUT), jnp.float32)], axis=1)


def _fin_body(p0_ref, p1_ref, h2_ref, dinv_ref, b2_ref, out_ref):
    d = dinv_ref[...]
    o = d * (p0_ref[...][:, :D_OUT] + p1_ref[...][:, :D_OUT]
             + h2_ref[...][:, :D_OUT]) + b2_ref[...]
    m = jnp.max(o, axis=1, keepdims=True)
    e = jnp.exp(o - m)
    lse = jnp.log(jnp.sum(e, axis=1, keepdims=True))
    out_ref[...] = o - m - lse


def _row_spec(dcols):
    return pl.BlockSpec((BM, dcols), lambda i: (i, 0))


def _full_spec(shape):
    return pl.BlockSpec(shape, lambda i: tuple(0 for _ in shape))


_GRID = (N_PAD // BM,)

_mm1 = pl.pallas_call(
    _mm1_body,
    grid=_GRID,
    in_specs=[_row_spec(D_IN), _full_spec((D_IN, D_HID)), _row_spec(1)],
    out_specs=[_row_spec(128)] * 2,
    out_shape=[jax.ShapeDtypeStruct((N_PAD, 128), jnp.float32)] * 2,
)

_mm2 = pl.pallas_call(
    _mm2_body,
    grid=_GRID,
    in_specs=[_row_spec(128)] * 4 +
             [_row_spec(1), _full_spec((1, D_HID)), _full_spec((D_HID, D_OUT))],
    out_specs=_row_spec(128),
    out_shape=jax.ShapeDtypeStruct((N_PAD, 128), jnp.float32),
)

_fin = pl.pallas_call(
    _fin_body,
    grid=_GRID,
    in_specs=[_row_spec(128)] * 3 + [_row_spec(1), _full_spec((1, D_OUT))],
    out_specs=_row_spec(D_OUT),
    out_shape=jax.ShapeDtypeStruct((N_PAD, D_OUT), jnp.float32),
)


def kernel(x, edge_index, W1, b1, W2, b2):
    src = edge_index[0].astype(jnp.int32)
    dst = edge_index[1].astype(jnp.int32)
    x_pad = jnp.pad(x, ((0, N_PAD - N), (0, 0)))

    deg_p, fsrc, fdst, cnts = _get_prep_kernel()(src, dst)
    fsrc = fsrc.reshape(NT * NB, FS)
    fdst = fdst.reshape(NT * NB, FS)
    deg = deg_p.sum(axis=0) + 1.0
    dinv = lax.rsqrt(deg)[:, None]

    def params(nf, f0a, f0b):
        return jnp.array([nf, f0a, f0b] + [0] * 13, jnp.int32)

    h1_0, h1_1 = _mm1(x_pad, W1, dinv)
    s1_0, s1_1 = _get_agg_kernel(8)(h1_0, h1_1, fsrc, fdst, cnts,
                                    params(32, 0, 0))
    h2 = _mm2(s1_0, s1_1, h1_0, h1_1, dinv, b1.reshape(1, D_HID), W2)
    p0, p1 = _get_agg_kernel(4)(h2, h2, fsrc, fdst, cnts, params(16, 0, 16))
    out = _fin(p0, p1, h2, dinv, b2.reshape(1, D_OUT))
    return out[:N]
